# baseline (speedup 1.0000x reference)
import functools

import jax
import jax.numpy as jnp
from jax import lax
from jax.experimental import pallas as pl
from jax.experimental.pallas import tpu as pltpu
from jax.experimental.pallas import tpu_sc as plsc

_D = 64
_C = 256
_NBUF = 2


def _sc_info():
    try:
        info = plsc.get_sparse_core_info()
        return info.num_cores, info.num_subcores
    except Exception:
        return 2, 16


@functools.lru_cache(maxsize=None)
def _build(N, S):
    NC, NS = _sc_info()
    NW = NC * NS
    assert N % _C == 0
    chunks_per_s = N // _C
    total_chunks = S * chunks_per_s
    assert total_chunks % (NW * _NBUF) == 0
    per_w = total_chunks // NW
    ngroups = per_w // _NBUF

    mesh = plsc.VectorSubcoreMesh(core_axis_name="c", subcore_axis_name="s")

    @functools.partial(
        pl.kernel,
        mesh=mesh,
        out_type=jax.ShapeDtypeStruct((S, _D, N), jnp.float32),
        scratch_types=[pltpu.VMEM((_C,), jnp.int32)] * _NBUF
        + [pltpu.VMEM((_C, _D), jnp.float32)] * _NBUF
        + [pltpu.VMEM((_D, _C), jnp.float32)] * _NBUF
        + [pltpu.SemaphoreType.DMA] * _NBUF
        + [pltpu.SemaphoreType.DMA] * _NBUF,
        compiler_params=pltpu.CompilerParams(
            use_tc_tiling_on_sc=False, needs_layout_passes=False
        ),
    )
    def gather_kernel(idx_hbm, table_hbm, out_hbm, *scratch):
        idx_v = scratch[:_NBUF]
        rows_v = scratch[_NBUF:2 * _NBUF]
        trans_v = scratch[2 * _NBUF:3 * _NBUF]
        gsems = scratch[3 * _NBUF:4 * _NBUF]
        ssems = scratch[4 * _NBUF:]
        wid = lax.axis_index("s") * NC + lax.axis_index("c")
        base_chunk = wid * per_w

        iota16 = lax.iota(jnp.int32, 16)
        row_vecs = [c16 * 16 + iota16 for c16 in range(16)]
        zeros16 = jnp.zeros((16,), jnp.int32)

        def start_gather(i, b):
            g = base_chunk + i
            pltpu.sync_copy(idx_hbm.at[pl.ds(g * _C, _C)], idx_v[b])
            pltpu.async_copy(table_hbm.at[idx_v[b]], rows_v[b], gsems[b])

        def wait_gather(b):
            pltpu.make_async_copy(
                table_hbm.at[idx_v[b]], rows_v[b], gsems[b]
            ).wait()

        def dst_of(i):
            g = base_chunk + i
            s = g // chunks_per_s
            b0 = (g % chunks_per_s) * _C
            return out_hbm.at[s, :, pl.ds(b0, _C)]

        def transpose(b):
            rows = rows_v[b]
            trans = trans_v[b]

            def jbody(j, _):
                col_j = zeros16 + j
                for c16 in range(16):
                    v = plsc.load_gather(rows, [row_vecs[c16], col_j])
                    trans[j, pl.ds(c16 * 16, 16)] = v
                return 0

            lax.fori_loop(0, _D, jbody, 0)

        for b in range(_NBUF):
            start_gather(b, b)

        def group_body(grp, _):
            for b in range(_NBUF):
                i = grp * _NBUF + b
                wait_gather(b)

                @pl.when(grp > 0)
                def _():
                    pltpu.make_async_copy(
                        trans_v[b], dst_of(i - _NBUF), ssems[b]
                    ).wait()

                transpose(b)
                pltpu.async_copy(trans_v[b], dst_of(i), ssems[b])

                @pl.when(grp < ngroups - 1)
                def _():
                    start_gather(i + _NBUF, b)

            return 0

        lax.fori_loop(0, ngroups, group_body, 0)

        for b in range(_NBUF):
            i = (ngroups - 1) * _NBUF + b
            pltpu.make_async_copy(trans_v[b], dst_of(i), ssems[b]).wait()

    return gather_kernel


def kernel(idxs, syn0):
    N, S = idxs.shape
    flat_t = jnp.transpose(idxs).reshape(-1).astype(jnp.int32)
    out = _build(N, S)(flat_t, syn0)
    return out.transpose(2, 0, 1)


# speedup vs baseline: 1.0252x; 1.0252x over previous
import functools

import jax
import jax.numpy as jnp
from jax import lax
from jax.experimental import pallas as pl
from jax.experimental.pallas import tpu as pltpu
from jax.experimental.pallas import tpu_sc as plsc

_D = 64
_C = 256
_NBUF = 2


def _sc_info():
    try:
        info = plsc.get_sparse_core_info()
        return info.num_cores, info.num_subcores
    except Exception:
        return 2, 16


@functools.lru_cache(maxsize=None)
def _build(N, S):
    NC, NS = _sc_info()
    NW = NC * NS
    assert N % _C == 0
    chunks_per_s = N // _C
    total_chunks = S * chunks_per_s
    assert total_chunks % (NW * _NBUF) == 0
    per_w = total_chunks // NW
    ngroups = per_w // _NBUF

    mesh = plsc.VectorSubcoreMesh(core_axis_name="c", subcore_axis_name="s")

    @functools.partial(
        pl.kernel,
        mesh=mesh,
        out_type=jax.ShapeDtypeStruct((S, _D, N), jnp.float32),
        scratch_types=[pltpu.VMEM((per_w * _C,), jnp.int32)]
        + [pltpu.VMEM((_C, _D), jnp.float32)] * _NBUF
        + [pltpu.VMEM((_D, _C), jnp.float32)] * _NBUF
        + [pltpu.SemaphoreType.DMA] * _NBUF
        + [pltpu.SemaphoreType.DMA] * _NBUF,
        compiler_params=pltpu.CompilerParams(
            use_tc_tiling_on_sc=False, needs_layout_passes=False
        ),
    )
    def gather_kernel(idx_hbm, table_hbm, out_hbm, idx_v, *scratch):
        rows_v = scratch[:_NBUF]
        trans_v = scratch[_NBUF:2 * _NBUF]
        gsems = scratch[2 * _NBUF:3 * _NBUF]
        ssems = scratch[3 * _NBUF:]
        wid = lax.axis_index("s") * NC + lax.axis_index("c")
        base_chunk = wid * per_w

        iota16 = lax.iota(jnp.int32, 16)
        row_vecs = [c16 * 16 + iota16 for c16 in range(16)]
        zeros16 = jnp.zeros((16,), jnp.int32)

        pltpu.sync_copy(idx_hbm.at[pl.ds(base_chunk * _C, per_w * _C)], idx_v)

        def gather_args(i, b):
            idx_slice = idx_v.at[pl.ds(i * _C, _C)]
            return table_hbm.at[idx_slice], rows_v[b], gsems[b]

        def dst_of(i):
            g = base_chunk + i
            s = g // chunks_per_s
            b0 = (g % chunks_per_s) * _C
            return out_hbm.at[s, :, pl.ds(b0, _C)]

        def transpose(b):
            rows = rows_v[b]
            trans = trans_v[b]

            def jq_body(jq, _):
                for dj in range(4):
                    j = jq * 4 + dj
                    col_j = zeros16 + j
                    for c16 in range(16):
                        v = plsc.load_gather(rows, [row_vecs[c16], col_j])
                        trans[j, pl.ds(c16 * 16, 16)] = v
                return 0

            lax.fori_loop(0, _D // 4, jq_body, 0)

        for b in range(_NBUF):
            pltpu.async_copy(*gather_args(b, b))

        def group_body(grp, _):
            for b in range(_NBUF):
                i = grp * _NBUF + b
                pltpu.make_async_copy(*gather_args(i, b)).wait()

                @pl.when(grp > 0)
                def _():
                    pltpu.make_async_copy(
                        trans_v[b], dst_of(i - _NBUF), ssems[b]
                    ).wait()

                transpose(b)
                pltpu.async_copy(trans_v[b], dst_of(i), ssems[b])

                @pl.when(grp < ngroups - 1)
                def _():
                    pltpu.async_copy(*gather_args(i + _NBUF, b))

            return 0

        lax.fori_loop(0, ngroups, group_body, 0)

        for b in range(_NBUF):
            i = (ngroups - 1) * _NBUF + b
            pltpu.make_async_copy(trans_v[b], dst_of(i), ssems[b]).wait()

    return gather_kernel


def kernel(idxs, syn0):
    N, S = idxs.shape
    flat_t = jnp.transpose(idxs).reshape(-1).astype(jnp.int32)
    out = _build(N, S)(flat_t, syn0)
    return out.transpose(2, 0, 1)


# speedup vs baseline: 1.7033x; 1.6615x over previous
import functools

import jax
import jax.numpy as jnp
from jax import lax
from jax.experimental import pallas as pl
from jax.experimental.pallas import tpu as pltpu
from jax.experimental.pallas import tpu_sc as plsc

_D = 64
_C = 256
_NBUF = 2


def _sc_info():
    try:
        info = plsc.get_sparse_core_info()
        return info.num_cores, info.num_subcores
    except Exception:
        return 2, 16


@functools.lru_cache(maxsize=None)
def _build(N, S):
    NC, NS = _sc_info()
    NW = NC * NS
    assert N % _C == 0
    chunks_per_s = N // _C
    total_chunks = S * chunks_per_s
    assert total_chunks % (NW * _NBUF) == 0
    per_w = total_chunks // NW
    ngroups = per_w // _NBUF

    mesh = plsc.VectorSubcoreMesh(core_axis_name="c", subcore_axis_name="s")

    @functools.partial(
        pl.kernel,
        mesh=mesh,
        out_type=jax.ShapeDtypeStruct((S, _D, N), jnp.float32),
        scratch_types=[pltpu.VMEM((per_w * _C,), jnp.int32)]
        + [pltpu.VMEM((_C, _D), jnp.float32)] * _NBUF
        + [pltpu.VMEM((_D, _C), jnp.float32)] * _NBUF
        + [pltpu.SemaphoreType.DMA] * _NBUF
        + [pltpu.SemaphoreType.DMA] * _NBUF,
        compiler_params=pltpu.CompilerParams(
            use_tc_tiling_on_sc=False, needs_layout_passes=False
        ),
    )
    def gather_kernel(idx_hbm, table_hbm, out_hbm, idx_v, *scratch):
        rows_v = scratch[:_NBUF]
        trans_v = scratch[_NBUF:2 * _NBUF]
        gsems = scratch[2 * _NBUF:3 * _NBUF]
        ssems = scratch[3 * _NBUF:]
        wid = lax.axis_index("s") * NC + lax.axis_index("c")
        base_chunk = wid * per_w

        iota16 = lax.iota(jnp.int32, 16)
        row_vecs = [c16 * 16 + iota16 for c16 in range(16)]
        zeros16 = jnp.zeros((16,), jnp.int32)

        pltpu.sync_copy(idx_hbm.at[pl.ds(base_chunk * _C, per_w * _C)], idx_v)

        def gather_args(i, b):
            idx_slice = idx_v.at[pl.ds(i * _C, _C)]
            return table_hbm.at[idx_slice], rows_v[b], gsems[b]

        def dst_of(i):
            g = base_chunk + i
            s = g // chunks_per_s
            b0 = (g % chunks_per_s) * _C
            return out_hbm.at[s, :, pl.ds(b0, _C)]

        def transpose(b):
            rows = rows_v[b]
            trans = trans_v[b]

            def jq_body(jq, _):
                for dj in range(4):
                    j = jq * 4 + dj
                    col_j = (iota16 + j) & (_D - 1)
                    for c16 in range(16):
                        v = plsc.load_gather(rows, [row_vecs[c16], col_j])
                        plsc.store_scatter(trans, [col_j, row_vecs[c16]], v)
                return 0

            lax.fori_loop(0, _D // 4, jq_body, 0)

        for b in range(_NBUF):
            pltpu.async_copy(*gather_args(b, b))

        def group_body(grp, _):
            for b in range(_NBUF):
                i = grp * _NBUF + b
                pltpu.make_async_copy(*gather_args(i, b)).wait()

                @pl.when(grp > 0)
                def _():
                    pltpu.make_async_copy(
                        trans_v[b], dst_of(i - _NBUF), ssems[b]
                    ).wait()

                transpose(b)
                pltpu.async_copy(trans_v[b], dst_of(i), ssems[b])

                @pl.when(grp < ngroups - 1)
                def _():
                    pltpu.async_copy(*gather_args(i + _NBUF, b))

            return 0

        lax.fori_loop(0, ngroups, group_body, 0)

        for b in range(_NBUF):
            i = (ngroups - 1) * _NBUF + b
            pltpu.make_async_copy(trans_v[b], dst_of(i), ssems[b]).wait()

    return gather_kernel


def kernel(idxs, syn0):
    N, S = idxs.shape
    flat_t = jnp.transpose(idxs).reshape(-1).astype(jnp.int32)
    out = _build(N, S)(flat_t, syn0)
    return out.transpose(2, 0, 1)


# speedup vs baseline: 2.0376x; 1.1962x over previous
import functools

import jax
import jax.numpy as jnp
from jax import lax
from jax.experimental import pallas as pl
from jax.experimental.pallas import tpu as pltpu
from jax.experimental.pallas import tpu_sc as plsc

_D = 64
_C = 256
_NBUF = 2


def _sc_info():
    try:
        info = plsc.get_sparse_core_info()
        return info.num_cores, info.num_subcores
    except Exception:
        return 2, 16


@functools.lru_cache(maxsize=None)
def _build(N, S):
    NC, NS = _sc_info()
    NW = NC * NS
    assert N % _C == 0
    chunks_per_s = N // _C
    total_chunks = S * chunks_per_s
    assert total_chunks % (NW * _NBUF) == 0
    per_w = total_chunks // NW
    ngroups = per_w // _NBUF

    mesh = plsc.VectorSubcoreMesh(core_axis_name="c", subcore_axis_name="s")

    @functools.partial(
        pl.kernel,
        mesh=mesh,
        out_type=jax.ShapeDtypeStruct((S, _D, N), jnp.float32),
        scratch_types=[pltpu.VMEM((per_w * _C,), jnp.int32)]
        + [pltpu.VMEM((_C,), jnp.int32)] * _NBUF
        + [pltpu.VMEM((_C, 2 * _D), jnp.float32)] * _NBUF
        + [pltpu.VMEM((_D, _C), jnp.float32)] * _NBUF
        + [pltpu.SemaphoreType.DMA] * _NBUF
        + [pltpu.SemaphoreType.DMA] * _NBUF,
        compiler_params=pltpu.CompilerParams(
            use_tc_tiling_on_sc=True, needs_layout_passes=False
        ),
    )
    def gather_kernel(idx_hbm, table_hbm, out_hbm, idx_v, *scratch):
        ibig_v = scratch[:_NBUF]
        rows_v = scratch[_NBUF:2 * _NBUF]
        trans_v = scratch[2 * _NBUF:3 * _NBUF]
        gsems = scratch[3 * _NBUF:4 * _NBUF]
        ssems = scratch[4 * _NBUF:]
        wid = lax.axis_index("s") * NC + lax.axis_index("c")
        base_chunk = wid * per_w

        iota16 = lax.iota(jnp.int32, 16)
        row_vecs = [c16 * 16 + iota16 for c16 in range(16)]
        zeros16 = jnp.zeros((16,), jnp.int32)

        pltpu.sync_copy(idx_hbm.at[pl.ds(base_chunk * _C, per_w * _C)], idx_v)

        def start_gather(i, b):
            for c16 in range(_C // 16):
                iv = idx_v[pl.ds(i * _C + c16 * 16, 16)]
                ibig_v[b][pl.ds(c16 * 16, 16)] = iv >> 1
            pltpu.async_copy(table_hbm.at[ibig_v[b]], rows_v[b], gsems[b])

        def wait_gather(b):
            pltpu.make_async_copy(
                table_hbm.at[ibig_v[b]], rows_v[b], gsems[b]
            ).wait()

        def dst_of(i):
            g = base_chunk + i
            s = g // chunks_per_s
            b0 = (g % chunks_per_s) * _C
            return out_hbm.at[s, :, pl.ds(b0, _C)]

        def transpose(i, b):
            rows = rows_v[b]
            trans = trans_v[b]
            par64 = []
            for c16 in range(16):
                iv = idx_v[pl.ds(i * _C + c16 * 16, 16)]
                par64.append((iv & 1) << 6)

            def jq_body(jq, _):
                for dj in range(4):
                    j = jq * 4 + dj
                    col_j = (iota16 + j) & (_D - 1)
                    for c16 in range(16):
                        v = plsc.load_gather(
                            rows, [row_vecs[c16], col_j + par64[c16]]
                        )
                        plsc.store_scatter(trans, [col_j, row_vecs[c16]], v)
                return 0

            lax.fori_loop(0, _D // 4, jq_body, 0)

        for b in range(_NBUF):
            start_gather(b, b)

        def group_body(grp, _):
            for b in range(_NBUF):
                i = grp * _NBUF + b
                wait_gather(b)

                @pl.when(grp > 0)
                def _():
                    pltpu.make_async_copy(
                        trans_v[b], dst_of(i - _NBUF), ssems[b]
                    ).wait()

                transpose(i, b)
                pltpu.async_copy(trans_v[b], dst_of(i), ssems[b])

                @pl.when(grp < ngroups - 1)
                def _():
                    start_gather(i + _NBUF, b)

            return 0

        lax.fori_loop(0, ngroups, group_body, 0)

        for b in range(_NBUF):
            i = (ngroups - 1) * _NBUF + b
            pltpu.make_async_copy(trans_v[b], dst_of(i), ssems[b]).wait()

    return gather_kernel


def kernel(idxs, syn0):
    N, S = idxs.shape
    flat_t = jnp.transpose(idxs).reshape(-1).astype(jnp.int32)
    syn2 = syn0.reshape(syn0.shape[0] // 2, 2 * syn0.shape[1])
    out = _build(N, S)(flat_t, syn2)
    return out.transpose(2, 0, 1)
